# baseline (speedup 1.0000x reference)
import jax
import jax.numpy as jnp
from jax.experimental import pallas as pl
from jax.experimental.pallas import tpu as pltpu

_NT = 512
_KT = 1024
_SUB = 256


def _vq_loss_kernel(maxlen_ref, x_ref, c_ref, out_ref, acc_ref):
    i = pl.program_id(0)
    j = pl.program_id(1)
    nk = pl.num_programs(1)
    max_ilen = maxlen_ref[0]
    t_dim = maxlen_ref[1]

    t0 = (i * _NT) % t_dim
    tile_active = t0 < max_ilen

    @pl.when(jnp.logical_and(i == 0, j == 0))
    def _init_out():
        out_ref[0, 0] = 0.0

    @pl.when(tile_active)
    def _compute():
        x = x_ref[...]
        acc = jnp.full((_NT, 1), jnp.inf, dtype=jnp.float32)
        for s in range(_KT // _SUB):
            ct = c_ref[:, s * _SUB:(s + 1) * _SUB]
            part = jnp.dot(x, ct, preferred_element_type=jnp.float32)
            acc = jnp.minimum(acc, jnp.min(part, axis=1, keepdims=True))

        @pl.when(j == 0)
        def _first():
            acc_ref[...] = acc

        @pl.when(j != 0)
        def _rest():
            acc_ref[...] = jnp.minimum(acc_ref[...], acc)

        @pl.when(j == nk - 1)
        def _finish():
            x32 = x.astype(jnp.float32)
            x_sq = jnp.sum(x32 * x32, axis=1, keepdims=True) - 2.0
            minv = acc_ref[...] + x_sq
            t_local = t0 + jax.lax.broadcasted_iota(jnp.int32, (_NT, 1), 0)
            masked = jnp.where(t_local < max_ilen, minv, 0.0)
            out_ref[0, 0] += jnp.sum(masked)


def kernel(xs, ilens, codebook):
    b, t, d = xs.shape
    k = codebook.shape[0]
    n = b * t
    flat = xs.reshape(n, d)

    c_sq = jnp.sum(codebook.astype(jnp.float32) ** 2, axis=1)
    c_hi = c_sq.astype(jnp.bfloat16)
    c_lo = (c_sq - c_hi.astype(jnp.float32)).astype(jnp.bfloat16)
    c_aug = jnp.concatenate(
        [(-2.0 * codebook.T).astype(jnp.bfloat16),
         c_hi[None, :], c_lo[None, :]], axis=0)
    x_aug = jnp.concatenate(
        [flat.astype(jnp.bfloat16),
         jnp.ones((n, 2), dtype=jnp.bfloat16)], axis=1)

    max_ilen = jnp.max(ilens)
    scalars = jnp.stack([max_ilen, jnp.int32(t)])

    total = pl.pallas_call(
        _vq_loss_kernel,
        grid=(n // _NT, k // _KT),
        in_specs=[
            pl.BlockSpec(memory_space=pltpu.SMEM),
            pl.BlockSpec((_NT, d + 2), lambda i, j: (i, 0)),
            pl.BlockSpec((d + 2, _KT), lambda i, j: (0, j)),
        ],
        out_specs=pl.BlockSpec((1, 1), lambda i, j: (0, 0),
                               memory_space=pltpu.SMEM),
        out_shape=jax.ShapeDtypeStruct((1, 1), jnp.float32),
        scratch_shapes=[pltpu.VMEM((_NT, 1), jnp.float32)],
        compiler_params=pltpu.CompilerParams(
            dimension_semantics=("arbitrary", "arbitrary")),
    )(scalars, x_aug, c_aug)

    count = jnp.float32(b * d) * max_ilen.astype(jnp.float32)
    commit_loss = total[0, 0] / count
    loss = 0.25 * commit_loss
    return (loss, commit_loss)


# speedup vs baseline: 1.5877x; 1.5877x over previous
import jax
import jax.numpy as jnp
from jax.experimental import pallas as pl
from jax.experimental.pallas import tpu as pltpu

_NT = 512
_KT = 2048
_L = 128


def _vq_loss_kernel(maxlen_ref, x_ref, c_ref, out_ref, acc_ref, csq_ref):
    i = pl.program_id(0)
    j = pl.program_id(1)
    nk = pl.num_programs(1)
    max_ilen = maxlen_ref[0]
    t_dim = maxlen_ref[1]

    t0 = (i * _NT) % t_dim
    tile_active = t0 < max_ilen

    @pl.when(jnp.logical_and(i == 0, j == 0))
    def _init_out():
        out_ref[0, 0] = 0.0

    @pl.when(tile_active)
    def _compute():
        x = x_ref[...]
        ct = c_ref[...]

        @pl.when(i == 0)
        def _fill_csq():
            c32 = ct.astype(jnp.float32)
            csq_ref[:, pl.ds(j * _KT, _KT)] = (
                0.25 * jnp.sum(c32 * c32, axis=0, keepdims=True))

        part = jnp.dot(x, ct, preferred_element_type=jnp.float32)
        csq = csq_ref[:, pl.ds(j * _KT, _KT)]

        m = part[:, 0:_L] + csq[:, 0:_L]
        for g in range(1, _KT // _L):
            sl = slice(g * _L, (g + 1) * _L)
            m = jnp.minimum(m, part[:, sl] + csq[:, sl])

        @pl.when(j == 0)
        def _first():
            acc_ref[...] = m

        @pl.when(j != 0)
        def _rest():
            acc_ref[...] = jnp.minimum(acc_ref[...], m)

        @pl.when(j == nk - 1)
        def _finish():
            x32 = x.astype(jnp.float32)
            x_sq = jnp.sum(x32 * x32, axis=1, keepdims=True)
            minv = jnp.min(acc_ref[...], axis=1, keepdims=True) + x_sq
            t_local = t0 + jax.lax.broadcasted_iota(jnp.int32, (_NT, 1), 0)
            masked = jnp.where(t_local < max_ilen, minv, 0.0)
            out_ref[0, 0] += jnp.sum(masked)


def kernel(xs, ilens, codebook):
    b, t, d = xs.shape
    k = codebook.shape[0]
    n = b * t
    flat = xs.reshape(n, d).astype(jnp.bfloat16)
    c_t = (-2.0 * codebook.T).astype(jnp.bfloat16)

    max_ilen = jnp.max(ilens)
    scalars = jnp.stack([max_ilen, jnp.int32(t)])

    total = pl.pallas_call(
        _vq_loss_kernel,
        grid=(n // _NT, k // _KT),
        in_specs=[
            pl.BlockSpec(memory_space=pltpu.SMEM),
            pl.BlockSpec((_NT, d), lambda i, j: (i, 0)),
            pl.BlockSpec((d, _KT), lambda i, j: (0, j)),
        ],
        out_specs=pl.BlockSpec((1, 1), lambda i, j: (0, 0),
                               memory_space=pltpu.SMEM),
        out_shape=jax.ShapeDtypeStruct((1, 1), jnp.float32),
        scratch_shapes=[
            pltpu.VMEM((_NT, _L), jnp.float32),
            pltpu.VMEM((1, k), jnp.float32),
        ],
        compiler_params=pltpu.CompilerParams(
            dimension_semantics=("arbitrary", "arbitrary")),
    )(scalars, flat, c_t)

    count = jnp.float32(b * d) * max_ilen.astype(jnp.float32)
    commit_loss = total[0, 0] / count
    loss = 0.25 * commit_loss
    return (loss, commit_loss)


# speedup vs baseline: 2.0937x; 1.3187x over previous
import jax
import jax.numpy as jnp
from jax.experimental import pallas as pl
from jax.experimental.pallas import tpu as pltpu

_NT = 1024
_KT = 4096
_L = 128


def _vq_loss_kernel(maxlen_ref, x_ref, c_ref, out_ref, acc_ref, csq_ref):
    i = pl.program_id(0)
    j = pl.program_id(1)
    nk = pl.num_programs(1)
    max_ilen = maxlen_ref[0]
    t_dim = maxlen_ref[1]

    t0 = (i * _NT) % t_dim
    tile_active = t0 < max_ilen

    @pl.when(jnp.logical_and(i == 0, j == 0))
    def _init_out():
        out_ref[0, 0] = 0.0

    @pl.when(tile_active)
    def _compute():
        x = x_ref[...]
        ct = c_ref[...]

        @pl.when(i == 0)
        def _fill_csq():
            c32 = ct.astype(jnp.float32)
            csq_ref[:, pl.ds(j * _KT, _KT)] = (
                0.25 * jnp.sum(c32 * c32, axis=0, keepdims=True))

        part = jnp.dot(x, ct, preferred_element_type=jnp.float32)
        csq = csq_ref[:, pl.ds(j * _KT, _KT)]

        m = part[:, 0:_L] + csq[:, 0:_L]
        for g in range(1, _KT // _L):
            sl = slice(g * _L, (g + 1) * _L)
            m = jnp.minimum(m, part[:, sl] + csq[:, sl])

        @pl.when(j == 0)
        def _first():
            acc_ref[...] = m

        @pl.when(j != 0)
        def _rest():
            acc_ref[...] = jnp.minimum(acc_ref[...], m)

        @pl.when(j == nk - 1)
        def _finish():
            x32 = x.astype(jnp.float32)
            x_sq = jnp.sum(x32 * x32, axis=1, keepdims=True)
            minv = jnp.min(acc_ref[...], axis=1, keepdims=True) + x_sq
            t_local = t0 + jax.lax.broadcasted_iota(jnp.int32, (_NT, 1), 0)
            masked = jnp.where(t_local < max_ilen, minv, 0.0)
            out_ref[0, 0] += jnp.sum(masked)


def kernel(xs, ilens, codebook):
    b, t, d = xs.shape
    k = codebook.shape[0]
    n = b * t
    flat = xs.reshape(n, d).astype(jnp.bfloat16)
    c_t = (-2.0 * codebook.T).astype(jnp.bfloat16)

    max_ilen = jnp.max(ilens)
    scalars = jnp.stack([max_ilen, jnp.int32(t)])

    total = pl.pallas_call(
        _vq_loss_kernel,
        grid=(n // _NT, k // _KT),
        in_specs=[
            pl.BlockSpec(memory_space=pltpu.SMEM),
            pl.BlockSpec((_NT, d), lambda i, j: (i, 0)),
            pl.BlockSpec((d, _KT), lambda i, j: (0, j)),
        ],
        out_specs=pl.BlockSpec((1, 1), lambda i, j: (0, 0),
                               memory_space=pltpu.SMEM),
        out_shape=jax.ShapeDtypeStruct((1, 1), jnp.float32),
        scratch_shapes=[
            pltpu.VMEM((_NT, _L), jnp.float32),
            pltpu.VMEM((1, k), jnp.float32),
        ],
        compiler_params=pltpu.CompilerParams(
            dimension_semantics=("arbitrary", "arbitrary")),
    )(scalars, flat, c_t)

    count = jnp.float32(b * d) * max_ilen.astype(jnp.float32)
    commit_loss = total[0, 0] / count
    loss = 0.25 * commit_loss
    return (loss, commit_loss)


# speedup vs baseline: 2.2000x; 1.0508x over previous
import jax
import jax.numpy as jnp
from jax.experimental import pallas as pl
from jax.experimental.pallas import tpu as pltpu

_NT = 1024
_KT = 8192
_L = 128


def _vq_loss_kernel(maxlen_ref, x_ref, c_ref, out_ref, acc_ref, csq_ref):
    i = pl.program_id(0)
    j = pl.program_id(1)
    nk = pl.num_programs(1)
    max_ilen = maxlen_ref[0]
    t_dim = maxlen_ref[1]

    t0 = (i * _NT) % t_dim
    tile_active = t0 < max_ilen

    @pl.when(jnp.logical_and(i == 0, j == 0))
    def _init_out():
        out_ref[0, 0] = 0.0

    @pl.when(tile_active)
    def _compute():
        x = x_ref[...]
        ct = c_ref[...]

        @pl.when(i == 0)
        def _fill_csq():
            c32 = ct.astype(jnp.float32)
            csq_ref[:, pl.ds(j * _KT, _KT)] = (
                0.25 * jnp.sum(c32 * c32, axis=0, keepdims=True))

        part = jnp.dot(x, ct, preferred_element_type=jnp.float32)
        csq = csq_ref[:, pl.ds(j * _KT, _KT)]

        m = part[:, 0:_L] + csq[:, 0:_L]
        for g in range(1, _KT // _L):
            sl = slice(g * _L, (g + 1) * _L)
            m = jnp.minimum(m, part[:, sl] + csq[:, sl])

        @pl.when(j == 0)
        def _first():
            acc_ref[...] = m

        @pl.when(j != 0)
        def _rest():
            acc_ref[...] = jnp.minimum(acc_ref[...], m)

        @pl.when(j == nk - 1)
        def _finish():
            x32 = x.astype(jnp.float32)
            x_sq = jnp.sum(x32 * x32, axis=1, keepdims=True)
            minv = jnp.min(acc_ref[...], axis=1, keepdims=True) + x_sq
            t_local = t0 + jax.lax.broadcasted_iota(jnp.int32, (_NT, 1), 0)
            masked = jnp.where(t_local < max_ilen, minv, 0.0)
            out_ref[0, 0] += jnp.sum(masked)


def kernel(xs, ilens, codebook):
    b, t, d = xs.shape
    k = codebook.shape[0]
    n = b * t
    flat = xs.reshape(n, d).astype(jnp.bfloat16)
    c_t = (-2.0 * codebook.T).astype(jnp.bfloat16)

    max_ilen = jnp.max(ilens)
    scalars = jnp.stack([max_ilen, jnp.int32(t)])

    total = pl.pallas_call(
        _vq_loss_kernel,
        grid=(n // _NT, k // _KT),
        in_specs=[
            pl.BlockSpec(memory_space=pltpu.SMEM),
            pl.BlockSpec((_NT, d), lambda i, j: (i, 0)),
            pl.BlockSpec((d, _KT), lambda i, j: (0, j)),
        ],
        out_specs=pl.BlockSpec((1, 1), lambda i, j: (0, 0),
                               memory_space=pltpu.SMEM),
        out_shape=jax.ShapeDtypeStruct((1, 1), jnp.float32),
        scratch_shapes=[
            pltpu.VMEM((_NT, _L), jnp.float32),
            pltpu.VMEM((1, k), jnp.float32),
        ],
        compiler_params=pltpu.CompilerParams(
            dimension_semantics=("arbitrary", "arbitrary")),
    )(scalars, flat, c_t)

    count = jnp.float32(b * d) * max_ilen.astype(jnp.float32)
    commit_loss = total[0, 0] / count
    loss = 0.25 * commit_loss
    return (loss, commit_loss)


# speedup vs baseline: 2.9161x; 1.3255x over previous
import jax
import jax.numpy as jnp
from jax.experimental import pallas as pl
from jax.experimental.pallas import tpu as pltpu

_NT = 1024
_KT = 8192
_L = 128

_TDIMS = (((1,), (1,)), ((), ()))


def _vq_loss_kernel(maxlen_ref, x_ref, c_ref, out_ref, acc_ref, csq_ref):
    i = pl.program_id(0)
    j = pl.program_id(1)
    nk = pl.num_programs(1)
    max_ilen = maxlen_ref[0]
    t_dim = maxlen_ref[1]

    t0 = (i * _NT) % t_dim
    tile_active = t0 < max_ilen

    @pl.when(jnp.logical_and(i == 0, j == 0))
    def _init_out():
        out_ref[0, 0] = 0.0

    @pl.when(tile_active)
    def _compute():
        x = x_ref[...]
        c = c_ref[...]

        @pl.when(i == 0)
        def _fill_csq():
            ones = jnp.ones((1, c.shape[1]), dtype=jnp.bfloat16)
            csq_ref[:, pl.ds(j * _KT, _KT)] = jax.lax.dot_general(
                ones, c * c, _TDIMS, preferred_element_type=jnp.float32)

        part = jax.lax.dot_general(
            x, c, _TDIMS, preferred_element_type=jnp.float32)
        csq = csq_ref[:, pl.ds(j * _KT, _KT)]

        m = part[:, 0:_L] + csq[:, 0:_L]
        for g in range(1, _KT // _L):
            sl = slice(g * _L, (g + 1) * _L)
            m = jnp.minimum(m, part[:, sl] + csq[:, sl])

        @pl.when(j == 0)
        def _first():
            acc_ref[...] = m

        @pl.when(j != 0)
        def _rest():
            acc_ref[...] = jnp.minimum(acc_ref[...], m)

        @pl.when(j == nk - 1)
        def _finish():
            x32 = x.astype(jnp.float32)
            x_sq = 0.25 * jnp.sum(x32 * x32, axis=1, keepdims=True)
            minv = jnp.min(acc_ref[...], axis=1, keepdims=True) + x_sq
            t_local = t0 + jax.lax.broadcasted_iota(jnp.int32, (_NT, 1), 0)
            masked = jnp.where(t_local < max_ilen, minv, 0.0)
            out_ref[0, 0] += jnp.sum(masked)


def kernel(xs, ilens, codebook):
    b, t, d = xs.shape
    k = codebook.shape[0]
    n = b * t
    x2 = (-2.0 * xs.reshape(n, d)).astype(jnp.bfloat16)
    cb = codebook.astype(jnp.bfloat16)

    max_ilen = jnp.max(ilens)
    scalars = jnp.stack([max_ilen, jnp.int32(t)])

    total = pl.pallas_call(
        _vq_loss_kernel,
        grid=(n // _NT, k // _KT),
        in_specs=[
            pl.BlockSpec(memory_space=pltpu.SMEM),
            pl.BlockSpec((_NT, d), lambda i, j: (i, 0)),
            pl.BlockSpec((_KT, d), lambda i, j: (j, 0)),
        ],
        out_specs=pl.BlockSpec((1, 1), lambda i, j: (0, 0),
                               memory_space=pltpu.SMEM),
        out_shape=jax.ShapeDtypeStruct((1, 1), jnp.float32),
        scratch_shapes=[
            pltpu.VMEM((_NT, _L), jnp.float32),
            pltpu.VMEM((1, k), jnp.float32),
        ],
        compiler_params=pltpu.CompilerParams(
            dimension_semantics=("arbitrary", "arbitrary")),
    )(scalars, x2, cb)

    count = jnp.float32(b * d) * max_ilen.astype(jnp.float32)
    commit_loss = total[0, 0] / count
    loss = 0.25 * commit_loss
    return (loss, commit_loss)


# speedup vs baseline: 3.5074x; 1.2027x over previous
import jax
import jax.numpy as jnp
from jax.experimental import pallas as pl
from jax.experimental.pallas import tpu as pltpu

_NT = 1024
_KT = 8192
_L = 128

_TDIMS = (((1,), (1,)), ((), ()))


def _vq_loss_kernel(maxlen_ref, x_ref, c_ref, out_ref, acc_ref, csq_ref,
                    cbf_ref):
    i = pl.program_id(0)
    j = pl.program_id(1)
    nk = pl.num_programs(1)
    max_ilen = maxlen_ref[0]
    t_dim = maxlen_ref[1]

    t0 = (i * _NT) % t_dim
    tile_active = t0 < max_ilen

    @pl.when(jnp.logical_and(i == 0, j == 0))
    def _init_out():
        out_ref[0, 0] = 0.0

    @pl.when(tile_active)
    def _compute():
        @pl.when(i == 0)
        def _fill_c():
            cb = c_ref[...].astype(jnp.bfloat16)
            cbf_ref[pl.ds(j * _KT, _KT), :] = cb
            ones = jnp.ones((1, cb.shape[1]), dtype=jnp.bfloat16)
            csq_ref[:, pl.ds(j * _KT, _KT)] = jax.lax.dot_general(
                ones, cb * cb, _TDIMS, preferred_element_type=jnp.float32)

        x = (-2.0 * x_ref[...]).astype(jnp.bfloat16)
        c = cbf_ref[pl.ds(j * _KT, _KT), :]

        part = jax.lax.dot_general(
            x, c, _TDIMS, preferred_element_type=jnp.float32)
        csq = csq_ref[:, pl.ds(j * _KT, _KT)]

        m = part[:, 0:_L] + csq[:, 0:_L]
        for g in range(1, _KT // _L):
            sl = slice(g * _L, (g + 1) * _L)
            m = jnp.minimum(m, part[:, sl] + csq[:, sl])

        @pl.when(j == 0)
        def _first():
            acc_ref[...] = m

        @pl.when(j != 0)
        def _rest():
            acc_ref[...] = jnp.minimum(acc_ref[...], m)

        @pl.when(j == nk - 1)
        def _finish():
            x32 = x_ref[...]
            x_sq = jnp.sum(x32 * x32, axis=1, keepdims=True)
            minv = jnp.min(acc_ref[...], axis=1, keepdims=True) + x_sq
            t_local = t0 + jax.lax.broadcasted_iota(jnp.int32, (_NT, 1), 0)
            masked = jnp.where(t_local < max_ilen, minv, 0.0)
            out_ref[0, 0] += jnp.sum(masked)


def kernel(xs, ilens, codebook):
    b, t, d = xs.shape
    k = codebook.shape[0]
    n = b * t
    flat = xs.reshape(n, d)

    max_ilen = jnp.max(ilens)
    scalars = jnp.stack([max_ilen, jnp.int32(t)])

    total = pl.pallas_call(
        _vq_loss_kernel,
        grid=(n // _NT, k // _KT),
        in_specs=[
            pl.BlockSpec(memory_space=pltpu.SMEM),
            pl.BlockSpec((_NT, d), lambda i, j: (i, 0)),
            pl.BlockSpec((_KT, d), lambda i, j: (j, 0)),
        ],
        out_specs=pl.BlockSpec((1, 1), lambda i, j: (0, 0),
                               memory_space=pltpu.SMEM),
        out_shape=jax.ShapeDtypeStruct((1, 1), jnp.float32),
        scratch_shapes=[
            pltpu.VMEM((_NT, _L), jnp.float32),
            pltpu.VMEM((1, k), jnp.float32),
            pltpu.VMEM((k, d), jnp.bfloat16),
        ],
        compiler_params=pltpu.CompilerParams(
            dimension_semantics=("arbitrary", "arbitrary")),
    )(scalars, flat, codebook)

    count = jnp.float32(b * d) * max_ilen.astype(jnp.float32)
    commit_loss = total[0, 0] / count
    loss = 0.25 * commit_loss
    return (loss, commit_loss)
